# Initial kernel scaffold; baseline (speedup 1.0000x reference)
import jax
import jax.numpy as jnp
from jax.experimental import pallas as pl


def kernel(rel_comp, rel_feat, ent_feat, rel_head_feat, rel_tail_feat, pattern_rel_ent, time_feat, W_ent, W_self, W_rel, W_time, g_edge_index, g_b_rel, g_inv, g_ori_idx, pg_edge_index, pg_rel, pg_ori_idx):
    raise NotImplementedError("write your pallas kernel here")


# speedup vs baseline: 3.7178x; 3.7178x over previous
import functools

import jax
import jax.numpy as jnp
from jax import lax
from jax.experimental import pallas as pl
from jax.experimental.pallas import tpu as pltpu
from jax.experimental.pallas import tpu_sc as plsc

NC = 2
NS = 16
NW = NC * NS
L = 16

D = 128
WEXT = 144
CB = 128
GB = 64
ZR = 64


def _prep_body(prel_ref, pgrel_ref, pgdst_ref, pgori_ref, relcomp_ref,
               relfeat_ref, relhead_ref, reltail_ref, wrel_ref,
               timefeat_ref, wtime_ref,
               v2_ref, initrel_ref, relemb_ref, timeemb_ref):
    f32 = jnp.float32
    prel = prel_ref[...]
    pgr = pgrel_ref[...]
    pgd = pgdst_ref[...]
    pgo = pgori_ref[...]
    epg = pgr.shape[0]
    npg = pgo.shape[0]
    nrelk = prel.shape[0]
    oh_rel = (pgr == lax.broadcasted_iota(jnp.int32, (epg, nrelk), 1)
              ).astype(f32)
    adst = (lax.broadcasted_iota(jnp.int32, (npg, epg), 0) == pgd
            ).astype(f32)
    s = jnp.dot(adst, oh_rel, preferred_element_type=f32)
    degp = jnp.sum(s, axis=1, keepdims=True)
    rpg = jnp.dot(s, prel, preferred_element_type=f32) / jnp.maximum(degp, 1.0)
    obs = pgo >= 0
    safe = jnp.where(obs, pgo, 0)
    nrel = relcomp_ref.shape[0]
    ohc = (safe == lax.broadcasted_iota(jnp.int32, (npg, nrel), 1)
           ).astype(f32)
    comp = jnp.dot(ohc, relcomp_ref[...], preferred_element_type=f32)
    rel_coef = jnp.where(obs, comp, rpg)
    heads = jnp.dot(rel_coef, relhead_ref[...], preferred_element_type=f32)
    tails = jnp.dot(rel_coef, reltail_ref[...], preferred_element_type=f32)
    init_rel = jnp.dot(rel_coef, relfeat_ref[...], preferred_element_type=f32)
    initrel_ref[...] = init_rel
    relemb_ref[...] = jnp.maximum(
        jnp.dot(init_rel, wrel_ref[...], preferred_element_type=f32), 0.0)
    timeemb_ref[...] = jnp.maximum(
        jnp.dot(timefeat_ref[...], wtime_ref[...], preferred_element_type=f32),
        0.0)
    both = jnp.concatenate([tails, heads], axis=0)
    ext = (lax.broadcasted_iota(jnp.int32, (2 * npg, WEXT - D), 1) == 0
           ).astype(f32)
    v2_ref[...] = jnp.concatenate([both, ext], axis=1)


def _k2_body(nrows, epad,
             v2_hbm, brel_hbm, inv_hbm, dst_hbm, ori_hbm, ent_hbm,
             hp_hbm, er_hbm,
             accum, zbuf, rows, erbuf, cbuf, dstbuf, brelbuf, invbuf,
             oribuf, safebuf, sem, sem2):
    cid = lax.axis_index("c")
    sid = lax.axis_index("s")
    wid = cid * NS + sid
    rows_per_tile = nrows // NS
    npg = v2_hbm.shape[0] // 2

    @pl.loop(0, ZR)
    def _zrow(r):
        for j in range(WEXT // L):
            zbuf[r, pl.ds(j * L, L)] = jnp.zeros((L,), jnp.float32)

    @pl.loop(0, rows_per_tile // ZR)
    def _zcp(k):
        pltpu.sync_copy(zbuf, accum.at[pl.ds(sid * rows_per_tile + k * ZR, ZR)])

    r_t = nrows // NW
    rbase = wid * r_t

    @pl.loop(0, r_t // GB)
    def _gath(k):
        off = rbase + k * GB
        pltpu.sync_copy(ori_hbm.at[pl.ds(off, GB)], oribuf)
        for j in range(GB // L):
            v = oribuf[pl.ds(j * L, L)]
            safebuf[pl.ds(j * L, L)] = jnp.maximum(v, 0)
        pltpu.async_copy(ent_hbm.at[safebuf], erbuf, sem2).wait()
        pltpu.sync_copy(erbuf, er_hbm.at[pl.ds(off, GB)])

    plsc.subcore_barrier()

    e_t = epad // NW
    ebase = wid * e_t

    @pl.loop(0, e_t // CB)
    def _edge(k):
        off = ebase + k * CB
        pltpu.sync_copy(brel_hbm.at[pl.ds(off, CB)], brelbuf)
        pltpu.sync_copy(inv_hbm.at[pl.ds(off, CB)], invbuf)
        pltpu.sync_copy(dst_hbm.at[pl.ds(off, CB)], dstbuf)
        for j in range(CB // L):
            b = brelbuf[pl.ds(j * L, L)]
            iv = invbuf[pl.ds(j * L, L)]
            cbuf[pl.ds(j * L, L)] = b + iv * npg
        pltpu.async_copy(v2_hbm.at[cbuf], rows, sem).wait()
        pltpu.sync_copy(rows, accum.at[dstbuf], add=True)

    plsc.subcore_barrier()

    pltpu.sync_copy(
        accum.at[pl.ds(sid * rows_per_tile, rows_per_tile)],
        hp_hbm.at[pl.ds(cid * nrows + sid * rows_per_tile, rows_per_tile)])


def _combine_body(hp0_ref, hp1_ref, er_ref, ori_ref, ie_ref, deg_ref):
    s = hp0_ref[...] + hp1_ref[...]
    deg = s[:, D:D + 1]
    h = s[:, :D] / jnp.maximum(deg, 1.0)
    obs = ori_ref[...] >= 0
    ie_ref[...] = jnp.where(obs, er_ref[...], h)
    deg_ref[...] = deg


def _k3_body(nrows, epad,
             ie_hbm, ir_hbm, src_hbm, brel_hbm, dst_hbm,
             aggp_hbm,
             accum, zbuf, iebuf, irbuf, srcbuf, brelbuf, dstbuf, sem, sem2):
    cid = lax.axis_index("c")
    sid = lax.axis_index("s")
    wid = cid * NS + sid
    rows_per_tile = nrows // NS

    @pl.loop(0, ZR)
    def _zrow(r):
        for j in range(D // L):
            zbuf[r, pl.ds(j * L, L)] = jnp.zeros((L,), jnp.float32)

    @pl.loop(0, rows_per_tile // ZR)
    def _zcp(k):
        pltpu.sync_copy(zbuf, accum.at[pl.ds(sid * rows_per_tile + k * ZR, ZR)])

    plsc.subcore_barrier()

    e_t = epad // NW
    ebase = wid * e_t

    @pl.loop(0, e_t // CB)
    def _edge(k):
        off = ebase + k * CB
        pltpu.sync_copy(src_hbm.at[pl.ds(off, CB)], srcbuf)
        pltpu.sync_copy(brel_hbm.at[pl.ds(off, CB)], brelbuf)
        pltpu.sync_copy(dst_hbm.at[pl.ds(off, CB)], dstbuf)
        cp_a = pltpu.async_copy(ie_hbm.at[srcbuf], iebuf, sem)
        cp_b = pltpu.async_copy(ir_hbm.at[brelbuf], irbuf, sem2)
        cp_a.wait()
        cp_b.wait()

        @pl.loop(0, CB)
        def _mul(r):
            for j in range(D // L):
                iebuf[r, pl.ds(j * L, L)] = (
                    iebuf[r, pl.ds(j * L, L)] * irbuf[r, pl.ds(j * L, L)])

        pltpu.sync_copy(iebuf, accum.at[dstbuf], add=True)

    plsc.subcore_barrier()

    pltpu.sync_copy(
        accum.at[pl.ds(sid * rows_per_tile, rows_per_tile)],
        aggp_hbm.at[pl.ds(cid * nrows + sid * rows_per_tile, rows_per_tile)])


def _final_body(a0_ref, a1_ref, deg_ref, ie_ref, went_ref, wself_ref, out_ref):
    f32 = jnp.float32
    a = (a0_ref[...] + a1_ref[...]) / jnp.maximum(deg_ref[...], 1.0)
    out_ref[...] = jnp.maximum(
        jnp.dot(a, went_ref[...], preferred_element_type=f32)
        + jnp.dot(ie_ref[...], wself_ref[...], preferred_element_type=f32),
        0.0)


def kernel(rel_comp, rel_feat, ent_feat, rel_head_feat, rel_tail_feat,
           pattern_rel_ent, time_feat, W_ent, W_self, W_rel, W_time,
           g_edge_index, g_b_rel, g_inv, g_ori_idx,
           pg_edge_index, pg_rel, pg_ori_idx):
    f32 = jnp.float32
    i32 = jnp.int32
    n = g_ori_idx.shape[0]
    e = g_b_rel.shape[0]
    npg = pg_ori_idx.shape[0]
    epg = pg_rel.shape[0]
    ntime = time_feat.shape[0]

    v2ext, init_rel, rel_emb, time_emb = pl.pallas_call(
        _prep_body,
        out_shape=[
            jax.ShapeDtypeStruct((2 * npg, WEXT), f32),
            jax.ShapeDtypeStruct((npg, D), f32),
            jax.ShapeDtypeStruct((npg, D), f32),
            jax.ShapeDtypeStruct((ntime, D), f32),
        ],
    )(pattern_rel_ent,
      pg_rel.astype(i32).reshape(epg, 1),
      pg_edge_index[1].astype(i32).reshape(1, epg),
      pg_ori_idx.astype(i32).reshape(npg, 1),
      rel_comp, rel_feat, rel_head_feat, rel_tail_feat, W_rel,
      time_feat, W_time)

    rows_per_tile_rows = NW * 320
    nrows = rows_per_tile_rows
    assert n <= nrows - 1
    chunk = NW * CB
    epad = ((e + chunk - 1) // chunk) * chunk
    src = jnp.pad(g_edge_index[0].astype(i32), (0, epad - e))
    dst = jnp.pad(g_edge_index[1].astype(i32), (0, epad - e),
                  constant_values=n)
    brel = jnp.pad(g_b_rel.astype(i32), (0, epad - e))
    inv = jnp.pad(g_inv.astype(i32), (0, epad - e))
    ori = jnp.pad(g_ori_idx.astype(i32), (0, nrows - n), constant_values=-1)

    mesh = plsc.VectorSubcoreMesh(core_axis_name="c", subcore_axis_name="s",
                                  num_cores=NC, num_subcores=NS)
    k2 = pl.kernel(
        functools.partial(_k2_body, nrows, epad),
        out_type=[
            jax.ShapeDtypeStruct((NC * nrows, WEXT), f32),
            jax.ShapeDtypeStruct((nrows, D), f32),
        ],
        mesh=mesh,
        compiler_params=pltpu.CompilerParams(use_tc_tiling_on_sc=False),
        scratch_types=[
            pltpu.VMEM_SHARED((nrows, WEXT), f32),
            pltpu.VMEM((ZR, WEXT), f32),
            pltpu.VMEM((CB, WEXT), f32),
            pltpu.VMEM((GB, D), f32),
            pltpu.VMEM((CB,), i32),
            pltpu.VMEM((CB,), i32),
            pltpu.VMEM((CB,), i32),
            pltpu.VMEM((CB,), i32),
            pltpu.VMEM((GB,), i32),
            pltpu.VMEM((GB,), i32),
            pltpu.SemaphoreType.DMA,
            pltpu.SemaphoreType.DMA,
        ],
    )
    hp, er = k2(v2ext, brel, inv, dst, ori, ent_feat)

    br = nrows // 16
    init_ent, deg = pl.pallas_call(
        _combine_body,
        grid=(16,),
        in_specs=[
            pl.BlockSpec((br, WEXT), lambda i: (i, 0)),
            pl.BlockSpec((br, WEXT), lambda i: (i, 0)),
            pl.BlockSpec((br, D), lambda i: (i, 0)),
            pl.BlockSpec((br, 1), lambda i: (i, 0)),
        ],
        out_specs=[
            pl.BlockSpec((br, D), lambda i: (i, 0)),
            pl.BlockSpec((br, 1), lambda i: (i, 0)),
        ],
        out_shape=[
            jax.ShapeDtypeStruct((nrows, D), f32),
            jax.ShapeDtypeStruct((nrows, 1), f32),
        ],
    )(hp[:nrows], hp[nrows:], er, ori.reshape(nrows, 1))

    k3 = pl.kernel(
        functools.partial(_k3_body, nrows, epad),
        out_type=jax.ShapeDtypeStruct((NC * nrows, D), f32),
        mesh=mesh,
        compiler_params=pltpu.CompilerParams(use_tc_tiling_on_sc=False),
        scratch_types=[
            pltpu.VMEM_SHARED((nrows, D), f32),
            pltpu.VMEM((ZR, D), f32),
            pltpu.VMEM((CB, D), f32),
            pltpu.VMEM((CB, D), f32),
            pltpu.VMEM((CB,), i32),
            pltpu.VMEM((CB,), i32),
            pltpu.VMEM((CB,), i32),
            pltpu.SemaphoreType.DMA,
            pltpu.SemaphoreType.DMA,
        ],
    )
    aggp = k3(init_ent, init_rel, src, brel, dst)

    ent_full = pl.pallas_call(
        _final_body,
        grid=(16,),
        in_specs=[
            pl.BlockSpec((br, D), lambda i: (i, 0)),
            pl.BlockSpec((br, D), lambda i: (i, 0)),
            pl.BlockSpec((br, 1), lambda i: (i, 0)),
            pl.BlockSpec((br, D), lambda i: (i, 0)),
            pl.BlockSpec((D, D), lambda i: (0, 0)),
            pl.BlockSpec((D, D), lambda i: (0, 0)),
        ],
        out_specs=pl.BlockSpec((br, D), lambda i: (i, 0)),
        out_shape=jax.ShapeDtypeStruct((nrows, D), f32),
    )(aggp[:nrows], aggp[nrows:], deg, init_ent, W_ent, W_self)

    return (ent_full[:n], rel_emb, time_emb)


# speedup vs baseline: 4.1625x; 1.1196x over previous
import functools

import jax
import jax.numpy as jnp
from jax import lax
from jax.experimental import pallas as pl
from jax.experimental.pallas import tpu as pltpu
from jax.experimental.pallas import tpu_sc as plsc

NC = 2
NS = 16
NW = NC * NS
L = 16

D = 128
WEXT = 144
CB = 64
GB = 64
ZR = 16


def _prep_body(prel_ref, pgrel_ref, pgdst_ref, pgori_ref, relcomp_ref,
               relfeat_ref, relhead_ref, reltail_ref, wrel_ref,
               timefeat_ref, wtime_ref,
               v2_ref, initrel_ref, relemb_ref, timeemb_ref):
    f32 = jnp.float32
    prel = prel_ref[...]
    pgr = pgrel_ref[...]
    pgd = pgdst_ref[...]
    pgo = pgori_ref[...]
    epg = pgr.shape[0]
    npg = pgo.shape[0]
    nrelk = prel.shape[0]
    oh_rel = (pgr == lax.broadcasted_iota(jnp.int32, (epg, nrelk), 1)
              ).astype(f32)
    adst = (lax.broadcasted_iota(jnp.int32, (npg, epg), 0) == pgd
            ).astype(f32)
    s = jnp.dot(adst, oh_rel, preferred_element_type=f32)
    degp = jnp.sum(s, axis=1, keepdims=True)
    rpg = jnp.dot(s, prel, preferred_element_type=f32) / jnp.maximum(degp, 1.0)
    obs = pgo >= 0
    safe = jnp.where(obs, pgo, 0)
    nrel = relcomp_ref.shape[0]
    ohc = (safe == lax.broadcasted_iota(jnp.int32, (npg, nrel), 1)
           ).astype(f32)
    comp = jnp.dot(ohc, relcomp_ref[...], preferred_element_type=f32)
    rel_coef = jnp.where(obs, comp, rpg)
    heads = jnp.dot(rel_coef, relhead_ref[...], preferred_element_type=f32)
    tails = jnp.dot(rel_coef, reltail_ref[...], preferred_element_type=f32)
    init_rel = jnp.dot(rel_coef, relfeat_ref[...], preferred_element_type=f32)
    initrel_ref[...] = init_rel
    relemb_ref[...] = jnp.maximum(
        jnp.dot(init_rel, wrel_ref[...], preferred_element_type=f32), 0.0)
    timeemb_ref[...] = jnp.maximum(
        jnp.dot(timefeat_ref[...], wtime_ref[...], preferred_element_type=f32),
        0.0)
    both = jnp.concatenate([tails, heads], axis=0)
    ext = (lax.broadcasted_iota(jnp.int32, (2 * npg, WEXT - D), 1) == 0
           ).astype(f32)
    v2_ref[...] = jnp.concatenate([both, ext], axis=1)


def _k2_body(nrows, epad,
             v2_hbm, idx3_hbm, ori_hbm, ent_hbm,
             hp_hbm, er_hbm,
             accum, zbuf, rows0, rows1, erbuf, ib0, ib1, cb0, cb1, db0, db1,
             oribuf, safebuf,
             semi0, semi1, semg0, semg1, sems0, sems1, seme):
    cid = lax.axis_index("c")
    sid = lax.axis_index("s")
    wid = cid * NS + sid
    rpt = nrows // NS
    npg = v2_hbm.shape[0] // 2
    nch = epad // (NW * CB)
    chbase = wid * nch
    ib = (ib0, ib1)
    cb = (cb0, cb1)
    db = (db0, db1)
    rows = (rows0, rows1)
    semi = (semi0, semi1)
    semg = (semg0, semg1)
    sems = (sems0, sems1)

    @pl.loop(0, ZR)
    def _zrow(r):
        for j in range(WEXT // L):
            zbuf[r, pl.ds(j * L, L)] = jnp.zeros((L,), jnp.float32)

    @pl.loop(0, rpt // ZR)
    def _zcp(k):
        pltpu.sync_copy(zbuf, accum.at[pl.ds(sid * rpt + k * ZR, ZR)])

    r_t = nrows // NW
    rbase = wid * r_t

    @pl.loop(0, r_t // GB)
    def _gath(k):
        off = rbase + k * GB
        pltpu.sync_copy(ori_hbm.at[pl.ds(off, GB)], oribuf)
        for j in range(GB // L):
            v = oribuf[pl.ds(j * L, L)]
            safebuf[pl.ds(j * L, L)] = jnp.maximum(v, 0)
        pltpu.async_copy(ent_hbm.at[safebuf], erbuf, seme).wait()
        pltpu.sync_copy(erbuf, er_hbm.at[pl.ds(off, GB)])

    plsc.subcore_barrier()

    def start_idx(b, ch):
        pltpu.async_copy(idx3_hbm.at[pl.ds((chbase + ch) * (3 * CB), 3 * CB)],
                         ib[b], semi[b])

    def wait_idx(b):
        pltpu.make_async_copy(idx3_hbm.at[pl.ds(0, 3 * CB)], ib[b],
                              semi[b]).wait()

    def wait_scat(b):
        pltpu.make_async_copy(rows[b], accum.at[db[b]], sems[b]).wait()

    start_idx(0, 0)
    start_idx(1, 1)

    @pl.loop(0, nch // 2)
    def _main(jj):
        descs = []
        for b in range(2):
            wait_idx(b)

            @pl.when(jj >= 1)
            def _(b=b):
                wait_scat(b)

            for j in range(CB // L):
                brelv = ib[b][pl.ds(j * L, L)]
                invv = ib[b][pl.ds(CB + j * L, L)]
                cb[b][pl.ds(j * L, L)] = brelv + invv * npg
                db[b][pl.ds(j * L, L)] = ib[b][pl.ds(2 * CB + j * L, L)]
            descs.append(pltpu.async_copy(v2_hbm.at[cb[b]], rows[b], semg[b]))
        for b in range(2):
            descs[b].wait()

            @pl.when(jj < nch // 2 - 1)
            def _(b=b):
                start_idx(b, 2 * jj + 2 + b)

            pltpu.async_copy(rows[b], accum.at[db[b]], sems[b], add=True)

    wait_scat(0)
    wait_scat(1)
    plsc.subcore_barrier()

    pltpu.sync_copy(
        accum.at[pl.ds(sid * rpt, rpt)],
        hp_hbm.at[pl.ds(cid * nrows + sid * rpt, rpt)])


def _combine_body(hp0_ref, hp1_ref, er_ref, ori_ref, ie_ref, deg_ref):
    s = hp0_ref[...] + hp1_ref[...]
    deg = s[:, D:D + 1]
    h = s[:, :D] / jnp.maximum(deg, 1.0)
    obs = ori_ref[...] >= 0
    ie_ref[...] = jnp.where(obs, er_ref[...], h)
    deg_ref[...] = deg


def _k3_body(nrows, epad,
             ie_hbm, ir_hbm, idx3_hbm,
             aggp_hbm,
             accum, zbuf, ib0, ib1, db0, db1, ieb0, ieb1, irb0, irb1,
             semi0, semi1, semg0, semg1, sems0, sems1):
    cid = lax.axis_index("c")
    sid = lax.axis_index("s")
    wid = cid * NS + sid
    rpt = nrows // NS
    nch = epad // (NW * CB)
    chbase = wid * nch
    ib = (ib0, ib1)
    db = (db0, db1)
    ieb = (ieb0, ieb1)
    irb = (irb0, irb1)
    semi = (semi0, semi1)
    semg = (semg0, semg1)
    sems = (sems0, sems1)

    @pl.loop(0, ZR)
    def _zrow(r):
        for j in range(D // L):
            zbuf[r, pl.ds(j * L, L)] = jnp.zeros((L,), jnp.float32)

    @pl.loop(0, rpt // ZR)
    def _zcp(k):
        pltpu.sync_copy(zbuf, accum.at[pl.ds(sid * rpt + k * ZR, ZR)])

    plsc.subcore_barrier()

    def start_idx(b, ch):
        pltpu.async_copy(idx3_hbm.at[pl.ds((chbase + ch) * (3 * CB), 3 * CB)],
                         ib[b], semi[b])

    def wait_idx(b):
        pltpu.make_async_copy(idx3_hbm.at[pl.ds(0, 3 * CB)], ib[b],
                              semi[b]).wait()

    def wait_scat(b):
        pltpu.make_async_copy(ieb[b], accum.at[db[b]], sems[b]).wait()

    start_idx(0, 0)
    start_idx(1, 1)

    @pl.loop(0, nch // 2)
    def _main(jj):
        descs = []
        for b in range(2):
            wait_idx(b)

            @pl.when(jj >= 1)
            def _(b=b):
                wait_scat(b)

            for j in range(CB // L):
                db[b][pl.ds(j * L, L)] = ib[b][pl.ds(2 * CB + j * L, L)]
            ga = pltpu.async_copy(ie_hbm.at[ib[b].at[pl.ds(0, CB)]],
                                  ieb[b], semg[b])
            gb = pltpu.async_copy(ir_hbm.at[ib[b].at[pl.ds(CB, CB)]],
                                  irb[b], semg[b])
            descs.append((ga, gb))
        for b in range(2):
            ga, gb = descs[b]
            ga.wait()
            gb.wait()

            @pl.when(jj < nch // 2 - 1)
            def _(b=b):
                start_idx(b, 2 * jj + 2 + b)

            @plsc.parallel_loop(0, CB, unroll=4)
            def _mul(r):
                for j in range(D // L):
                    ieb[b][r, pl.ds(j * L, L)] = (
                        ieb[b][r, pl.ds(j * L, L)] * irb[b][r, pl.ds(j * L, L)])

            pltpu.async_copy(ieb[b], accum.at[db[b]], sems[b], add=True)

    wait_scat(0)
    wait_scat(1)
    plsc.subcore_barrier()

    pltpu.sync_copy(
        accum.at[pl.ds(sid * rpt, rpt)],
        aggp_hbm.at[pl.ds(cid * nrows + sid * rpt, rpt)])


def _final_body(a0_ref, a1_ref, deg_ref, ie_ref, went_ref, wself_ref, out_ref):
    f32 = jnp.float32
    a = (a0_ref[...] + a1_ref[...]) / jnp.maximum(deg_ref[...], 1.0)
    out_ref[...] = jnp.maximum(
        jnp.dot(a, went_ref[...], preferred_element_type=f32)
        + jnp.dot(ie_ref[...], wself_ref[...], preferred_element_type=f32),
        0.0)


def kernel(rel_comp, rel_feat, ent_feat, rel_head_feat, rel_tail_feat,
           pattern_rel_ent, time_feat, W_ent, W_self, W_rel, W_time,
           g_edge_index, g_b_rel, g_inv, g_ori_idx,
           pg_edge_index, pg_rel, pg_ori_idx):
    f32 = jnp.float32
    i32 = jnp.int32
    n = g_ori_idx.shape[0]
    e = g_b_rel.shape[0]
    npg = pg_ori_idx.shape[0]
    epg = pg_rel.shape[0]
    ntime = time_feat.shape[0]

    v2ext, init_rel, rel_emb, time_emb = pl.pallas_call(
        _prep_body,
        out_shape=[
            jax.ShapeDtypeStruct((2 * npg, WEXT), f32),
            jax.ShapeDtypeStruct((npg, D), f32),
            jax.ShapeDtypeStruct((npg, D), f32),
            jax.ShapeDtypeStruct((ntime, D), f32),
        ],
    )(pattern_rel_ent,
      pg_rel.astype(i32).reshape(epg, 1),
      pg_edge_index[1].astype(i32).reshape(1, epg),
      pg_ori_idx.astype(i32).reshape(npg, 1),
      rel_comp, rel_feat, rel_head_feat, rel_tail_feat, W_rel,
      time_feat, W_time)

    nrows = NW * 320
    assert n <= nrows - 1
    chunk = 2 * NW * CB
    epad = ((e + chunk - 1) // chunk) * chunk
    src = jnp.pad(g_edge_index[0].astype(i32), (0, epad - e))
    dst = jnp.pad(g_edge_index[1].astype(i32), (0, epad - e),
                  constant_values=n)
    brel = jnp.pad(g_b_rel.astype(i32), (0, epad - e))
    inv = jnp.pad(g_inv.astype(i32), (0, epad - e))
    ori = jnp.pad(g_ori_idx.astype(i32), (0, nrows - n), constant_values=-1)
    idx3_k2 = jnp.stack([brel.reshape(-1, CB), inv.reshape(-1, CB),
                         dst.reshape(-1, CB)], axis=1).reshape(-1)
    idx3_k3 = jnp.stack([src.reshape(-1, CB), brel.reshape(-1, CB),
                         dst.reshape(-1, CB)], axis=1).reshape(-1)

    mesh = plsc.VectorSubcoreMesh(core_axis_name="c", subcore_axis_name="s",
                                  num_cores=NC, num_subcores=NS)
    k2 = pl.kernel(
        functools.partial(_k2_body, nrows, epad),
        out_type=[
            jax.ShapeDtypeStruct((NC * nrows, WEXT), f32),
            jax.ShapeDtypeStruct((nrows, D), f32),
        ],
        mesh=mesh,
        compiler_params=pltpu.CompilerParams(use_tc_tiling_on_sc=False),
        scratch_types=[
            pltpu.VMEM_SHARED((nrows, WEXT), f32),
            pltpu.VMEM((ZR, WEXT), f32),
            pltpu.VMEM((CB, WEXT), f32),
            pltpu.VMEM((CB, WEXT), f32),
            pltpu.VMEM((GB, D), f32),
            pltpu.VMEM((3 * CB,), i32),
            pltpu.VMEM((3 * CB,), i32),
            pltpu.VMEM((CB,), i32),
            pltpu.VMEM((CB,), i32),
            pltpu.VMEM((CB,), i32),
            pltpu.VMEM((CB,), i32),
            pltpu.VMEM((GB,), i32),
            pltpu.VMEM((GB,), i32),
            pltpu.SemaphoreType.DMA,
            pltpu.SemaphoreType.DMA,
            pltpu.SemaphoreType.DMA,
            pltpu.SemaphoreType.DMA,
            pltpu.SemaphoreType.DMA,
            pltpu.SemaphoreType.DMA,
            pltpu.SemaphoreType.DMA,
        ],
    )
    hp, er = k2(v2ext, idx3_k2, ori, ent_feat)

    br = nrows // 16
    init_ent, deg = pl.pallas_call(
        _combine_body,
        grid=(16,),
        in_specs=[
            pl.BlockSpec((br, WEXT), lambda i: (i, 0)),
            pl.BlockSpec((br, WEXT), lambda i: (i, 0)),
            pl.BlockSpec((br, D), lambda i: (i, 0)),
            pl.BlockSpec((br, 1), lambda i: (i, 0)),
        ],
        out_specs=[
            pl.BlockSpec((br, D), lambda i: (i, 0)),
            pl.BlockSpec((br, 1), lambda i: (i, 0)),
        ],
        out_shape=[
            jax.ShapeDtypeStruct((nrows, D), f32),
            jax.ShapeDtypeStruct((nrows, 1), f32),
        ],
    )(hp[:nrows], hp[nrows:], er, ori.reshape(nrows, 1))

    k3 = pl.kernel(
        functools.partial(_k3_body, nrows, epad),
        out_type=jax.ShapeDtypeStruct((NC * nrows, D), f32),
        mesh=mesh,
        compiler_params=pltpu.CompilerParams(use_tc_tiling_on_sc=False),
        scratch_types=[
            pltpu.VMEM_SHARED((nrows, D), f32),
            pltpu.VMEM((ZR, D), f32),
            pltpu.VMEM((3 * CB,), i32),
            pltpu.VMEM((3 * CB,), i32),
            pltpu.VMEM((CB,), i32),
            pltpu.VMEM((CB,), i32),
            pltpu.VMEM((CB, D), f32),
            pltpu.VMEM((CB, D), f32),
            pltpu.VMEM((CB, D), f32),
            pltpu.VMEM((CB, D), f32),
            pltpu.SemaphoreType.DMA,
            pltpu.SemaphoreType.DMA,
            pltpu.SemaphoreType.DMA,
            pltpu.SemaphoreType.DMA,
            pltpu.SemaphoreType.DMA,
            pltpu.SemaphoreType.DMA,
        ],
    )
    aggp = k3(init_ent, init_rel, idx3_k3)

    ent_full = pl.pallas_call(
        _final_body,
        grid=(16,),
        in_specs=[
            pl.BlockSpec((br, D), lambda i: (i, 0)),
            pl.BlockSpec((br, D), lambda i: (i, 0)),
            pl.BlockSpec((br, 1), lambda i: (i, 0)),
            pl.BlockSpec((br, D), lambda i: (i, 0)),
            pl.BlockSpec((D, D), lambda i: (0, 0)),
            pl.BlockSpec((D, D), lambda i: (0, 0)),
        ],
        out_specs=pl.BlockSpec((br, D), lambda i: (i, 0)),
        out_shape=jax.ShapeDtypeStruct((nrows, D), f32),
    )(aggp[:nrows], aggp[nrows:], deg, init_ent, W_ent, W_self)

    return (ent_full[:n], rel_emb, time_emb)
